# baseline (speedup 1.0000x reference)
import jax
import jax.numpy as jnp
from jax.experimental import pallas as pl
from jax.experimental.pallas import tpu as pltpu

_LOGIT_FLOOR = -1000000000.0
_TN = 256


def _silu(x):
    return x * jax.nn.sigmoid(x)


def _mega_kernel(hid_ref, amask_ref, w1_ref, w2_ref, tbl_ref, scale_ref,
                 bids_ref, nw_ref, nb_ref, aw_ref, ab_ref, nmask_ref,
                 amaskc_ref, tname_ref, targ_ref, out_ref,
                 pooled, x1, adapted, nlog, alog, sem):
    i = pl.program_id(0)
    B, S, H = hid_ref.shape
    P = w1_ref.shape[0]
    NB = tbl_ref.shape[0]
    NN = nmask_ref.shape[1]
    NA = amaskc_ref.shape[1]
    KT1 = P // _TN
    KT2 = P // _TN
    NT = NN // _TN
    AT = NA // _TN
    S1 = 1
    S2 = S1 + KT1
    S3 = S2 + KT2
    S4 = S3 + NT
    SF = S4 + AT - 1

    @pl.when(i == 0)
    def _gather():
        copies = []
        for b in range(B):
            s = jnp.sum(amask_ref[b, :])
            len_b = jnp.maximum(s, 1) - 1
            copies.append(pltpu.make_async_copy(
                hid_ref.at[b, pl.ds(len_b, 1), :],
                pooled.at[pl.ds(b, 1), :], sem))
        for c in copies:
            c.start()
        for c in copies:
            c.wait()

    @pl.when((i >= S1) & (i < S2))
    def _mm1():
        t = i - S1
        tile = _silu(jnp.dot(pooled[...], w1_ref[...],
                             preferred_element_type=jnp.float32))
        x1[pl.ds(t, 1)] = tile[None]

    @pl.when((i >= S2) & (i < S3))
    def _mm2():
        t = i - S2
        X1 = jnp.concatenate([x1[k] for k in range(KT1)], axis=1)
        f = _silu(jnp.dot(X1, w2_ref[...],
                          preferred_element_type=jnp.float32))
        onehot = jnp.where(jax.lax.broadcasted_iota(jnp.int32, (B, NB), 1)
                           == bids_ref[...], 1.0, 0.0).astype(jnp.float32)
        adpt = jnp.dot(onehot, tbl_ref[...],
                       preferred_element_type=jnp.float32)
        adapted[pl.ds(t, 1)] = (f + scale_ref[0, 0] * adpt)[None]

    @pl.when((i >= S3) & (i < S4))
    def _name():
        t = i - S3
        A = jnp.concatenate([adapted[k] for k in range(KT2)], axis=1)
        nlog[pl.ds(t, 1)] = jnp.dot(A, nw_ref[...],
                                    preferred_element_type=jnp.float32)[None]

    @pl.when(i >= S4)
    def _arg():
        t = i - S4
        A = jnp.concatenate([adapted[k] for k in range(KT2)], axis=1)
        alog[pl.ds(t, 1)] = jnp.dot(A, aw_ref[...],
                                    preferred_element_type=jnp.float32)[None]

    @pl.when(i == SF)
    def _finish():
        def head_loss(log_scr, ntiles, bias_ref, mask_ref, tgt_ref, ncls):
            logits = jnp.concatenate([log_scr[k] for k in range(ntiles)],
                                     axis=1) + bias_ref[...]
            mf = mask_ref[...]
            anyv = jnp.max(mf, axis=1, keepdims=True)
            eff = jnp.maximum(mf, 1.0 - anyv)
            lm = eff * logits + (1.0 - eff) * _LOGIT_FLOOR
            mx = jnp.max(lm, axis=1, keepdims=True)
            lse = jnp.log(jnp.sum(jnp.exp(lm - mx), axis=1)) + mx[:, 0]
            oh = jnp.where(jax.lax.broadcasted_iota(jnp.int32, (B, ncls), 1)
                           == tgt_ref[...], 1.0, 0.0).astype(jnp.float32)
            tgt = jnp.sum(lm * oh, axis=1)
            return jnp.mean(lse - tgt)

        nl = head_loss(nlog, NT, nb_ref, nmask_ref, tname_ref, NN)
        al = head_loss(alog, AT, ab_ref, amaskc_ref, targ_ref, NA)
        out_ref[0, 0] = nl + al


def kernel(hidden_states, W1, W2, adapter_table, adapter_scale, name_W,
           name_b, arg_W, arg_b, attention_mask, benchmark_ids,
           target_name_ids, target_argument_ids, name_candidate_masks,
           argument_candidate_masks):
    B, S, H = hidden_states.shape
    P = W1.shape[1]
    NB, _ = adapter_table.shape
    NN = name_W.shape[1]
    NA = arg_W.shape[1]
    KT1 = P // _TN
    KT2 = P // _TN
    NT = NN // _TN
    AT = NA // _TN
    nsteps = 1 + KT1 + KT2 + NT + AT
    S2 = 1 + KT1
    S3 = S2 + KT2
    S4 = S3 + NT

    out = pl.pallas_call(
        _mega_kernel,
        grid=(nsteps,),
        in_specs=[
            pl.BlockSpec(memory_space=pl.ANY),
            pl.BlockSpec((B, S), lambda i: (0, 0)),
            pl.BlockSpec((H, _TN), lambda i: (0, jnp.clip(i - 1, 0, KT1 - 1))),
            pl.BlockSpec((P, _TN), lambda i: (0, jnp.clip(i - S2, 0, KT2 - 1))),
            pl.BlockSpec((NB, _TN), lambda i: (0, jnp.clip(i - S2, 0, KT2 - 1))),
            pl.BlockSpec((1, 1), lambda i: (0, 0)),
            pl.BlockSpec((B, 1), lambda i: (0, 0)),
            pl.BlockSpec((P, _TN), lambda i: (0, jnp.clip(i - S3, 0, NT - 1))),
            pl.BlockSpec((1, NN), lambda i: (0, 0)),
            pl.BlockSpec((P, _TN), lambda i: (0, jnp.clip(i - S4, 0, AT - 1))),
            pl.BlockSpec((1, NA), lambda i: (0, 0)),
            pl.BlockSpec((B, NN), lambda i: (0, 0)),
            pl.BlockSpec((B, NA), lambda i: (0, 0)),
            pl.BlockSpec((B, 1), lambda i: (0, 0)),
            pl.BlockSpec((B, 1), lambda i: (0, 0)),
        ],
        out_specs=pl.BlockSpec(memory_space=pltpu.MemorySpace.SMEM),
        out_shape=jax.ShapeDtypeStruct((1, 1), jnp.float32),
        scratch_shapes=[
            pltpu.VMEM((B, H), jnp.float32),
            pltpu.VMEM((KT1, B, _TN), jnp.float32),
            pltpu.VMEM((KT2, B, _TN), jnp.float32),
            pltpu.VMEM((NT, B, _TN), jnp.float32),
            pltpu.VMEM((AT, B, _TN), jnp.float32),
            pltpu.SemaphoreType.DMA,
        ],
    )(
        hidden_states,
        attention_mask.astype(jnp.int32),
        W1, W2, adapter_table,
        adapter_scale.reshape(1, 1).astype(jnp.float32),
        benchmark_ids.reshape(B, 1).astype(jnp.int32),
        name_W, name_b.reshape(1, NN),
        arg_W, arg_b.reshape(1, NA),
        name_candidate_masks.astype(jnp.float32),
        argument_candidate_masks.astype(jnp.float32),
        target_name_ids.reshape(B, 1).astype(jnp.int32),
        target_argument_ids.reshape(B, 1).astype(jnp.int32),
    )
    return out[0, 0]


# speedup vs baseline: 1.1919x; 1.1919x over previous
import jax
import jax.numpy as jnp
from jax.experimental import pallas as pl
from jax.experimental.pallas import tpu as pltpu

_LOGIT_FLOOR = -1000000000.0
_TN = 256
_TA = 512


def _silu(x):
    return x * jax.nn.sigmoid(x)


def _mega_kernel(hid_ref, amask_ref, w1_ref, w2_ref, tbl_ref, scale_ref,
                 bids_ref, nw_ref, nb_ref, aw_ref, ab_ref, nmask_ref,
                 amaskc_ref, tname_ref, targ_ref, out_ref,
                 pooled, f_acc, adapted, nlog, alog, sem):
    i = pl.program_id(0)
    B, S, H = hid_ref.shape
    P = w1_ref.shape[0]
    NB = tbl_ref.shape[0]
    NN = nmask_ref.shape[1]
    NA = amaskc_ref.shape[1]
    KT = P // _TN
    AT = NA // _TA
    S2 = 1 + KT
    S3 = S2 + 1
    SF = S3 + AT - 1

    @pl.when(i == 0)
    def _gather():
        copies = []
        for b in range(B):
            s = jnp.sum(amask_ref[b, :])
            len_b = jnp.maximum(s, 1) - 1
            copies.append(pltpu.make_async_copy(
                hid_ref.at[b, pl.ds(len_b, 1), :],
                pooled.at[pl.ds(b, 1), :], sem))
        for c in copies:
            c.start()
        for c in copies:
            c.wait()

    @pl.when((i >= 1) & (i < S2))
    def _mlp():
        x1_t = _silu(jnp.dot(pooled[...], w1_ref[...],
                             preferred_element_type=jnp.float32))
        part = jnp.dot(x1_t, w2_ref[...], preferred_element_type=jnp.float32)

        @pl.when(i == 1)
        def _():
            f_acc[...] = part

        @pl.when(i > 1)
        def _():
            f_acc[...] += part

    @pl.when(i == S2)
    def _name():
        onehot = jnp.where(jax.lax.broadcasted_iota(jnp.int32, (B, NB), 1)
                           == bids_ref[...], 1.0, 0.0).astype(jnp.float32)
        adpt = jnp.dot(onehot, tbl_ref[...],
                       preferred_element_type=jnp.float32)
        A = _silu(f_acc[...]) + scale_ref[0, 0] * adpt
        adapted[...] = A
        nlog[...] = jnp.dot(A, nw_ref[...],
                            preferred_element_type=jnp.float32)

    for t in range(AT):
        @pl.when(i == S3 + t)
        def _arg(t=t):
            alog[:, t * _TA:(t + 1) * _TA] = jnp.dot(
                adapted[...], aw_ref[...], preferred_element_type=jnp.float32)

    @pl.when(i == SF)
    def _finish():
        def head_loss(logits, bias_ref, mask_ref, tgt_ref, ncls):
            logits = logits + bias_ref[...]
            mf = mask_ref[...]
            anyv = jnp.max(mf, axis=1, keepdims=True)
            eff = jnp.maximum(mf, 1.0 - anyv)
            lm = eff * logits + (1.0 - eff) * _LOGIT_FLOOR
            mx = jnp.max(lm, axis=1, keepdims=True)
            lse = jnp.log(jnp.sum(jnp.exp(lm - mx), axis=1)) + mx[:, 0]
            oh = jnp.where(jax.lax.broadcasted_iota(jnp.int32, (B, ncls), 1)
                           == tgt_ref[...], 1.0, 0.0).astype(jnp.float32)
            tgt = jnp.sum(lm * oh, axis=1)
            return jnp.mean(lse - tgt)

        nl = head_loss(nlog[...], nb_ref, nmask_ref, tname_ref, NN)
        al = head_loss(alog[...], ab_ref, amaskc_ref, targ_ref, NA)
        out_ref[0, 0] = nl + al


def kernel(hidden_states, W1, W2, adapter_table, adapter_scale, name_W,
           name_b, arg_W, arg_b, attention_mask, benchmark_ids,
           target_name_ids, target_argument_ids, name_candidate_masks,
           argument_candidate_masks):
    B, S, H = hidden_states.shape
    P = W1.shape[1]
    NB, _ = adapter_table.shape
    NN = name_W.shape[1]
    NA = arg_W.shape[1]
    KT = P // _TN
    AT = NA // _TA
    S2 = 1 + KT
    S3 = S2 + 1
    nsteps = S3 + AT

    out = pl.pallas_call(
        _mega_kernel,
        grid=(nsteps,),
        in_specs=[
            pl.BlockSpec(memory_space=pl.ANY),
            pl.BlockSpec((B, S), lambda i: (0, 0)),
            pl.BlockSpec((H, _TN), lambda i: (0, jnp.clip(i - 1, 0, KT - 1))),
            pl.BlockSpec((_TN, P), lambda i: (jnp.clip(i - 1, 0, KT - 1), 0)),
            pl.BlockSpec((NB, P), lambda i: (0, 0)),
            pl.BlockSpec((1, 1), lambda i: (0, 0)),
            pl.BlockSpec((B, 1), lambda i: (0, 0)),
            pl.BlockSpec((P, NN), lambda i: (0, 0)),
            pl.BlockSpec((1, NN), lambda i: (0, 0)),
            pl.BlockSpec((P, _TA), lambda i: (0, jnp.clip(i - S3, 0, AT - 1))),
            pl.BlockSpec((1, NA), lambda i: (0, 0)),
            pl.BlockSpec((B, NN), lambda i: (0, 0)),
            pl.BlockSpec((B, NA), lambda i: (0, 0)),
            pl.BlockSpec((B, 1), lambda i: (0, 0)),
            pl.BlockSpec((B, 1), lambda i: (0, 0)),
        ],
        out_specs=pl.BlockSpec(memory_space=pltpu.MemorySpace.SMEM),
        out_shape=jax.ShapeDtypeStruct((1, 1), jnp.float32),
        scratch_shapes=[
            pltpu.VMEM((B, H), jnp.float32),
            pltpu.VMEM((B, P), jnp.float32),
            pltpu.VMEM((B, P), jnp.float32),
            pltpu.VMEM((B, NN), jnp.float32),
            pltpu.VMEM((B, NA), jnp.float32),
            pltpu.SemaphoreType.DMA,
        ],
    )(
        hidden_states,
        attention_mask.astype(jnp.int32),
        W1, W2, adapter_table,
        adapter_scale.reshape(1, 1).astype(jnp.float32),
        benchmark_ids.reshape(B, 1).astype(jnp.int32),
        name_W, name_b.reshape(1, NN),
        arg_W, arg_b.reshape(1, NA),
        name_candidate_masks.astype(jnp.float32),
        argument_candidate_masks.astype(jnp.float32),
        target_name_ids.reshape(B, 1).astype(jnp.int32),
        target_argument_ids.reshape(B, 1).astype(jnp.int32),
    )
    return out[0, 0]


# speedup vs baseline: 1.1942x; 1.0020x over previous
import jax
import jax.numpy as jnp
from jax.experimental import pallas as pl
from jax.experimental.pallas import tpu as pltpu

_LOGIT_FLOOR = -1000000000.0
_TN = 256
_TA = 512


def _silu(x):
    return x * jax.nn.sigmoid(x)


def _mega_kernel(hid_ref, amask_ref, w1_ref, w2_ref, tbl_ref, scale_ref,
                 bids_ref, nw_ref, nb_ref, aw_ref, ab_ref, nmask_ref,
                 amaskc_ref, tname_ref, targ_ref, out_ref,
                 pooled, f_acc, adapted, nlog, alog, sem):
    i = pl.program_id(0)
    B, S, H = hid_ref.shape
    P = w1_ref.shape[0]
    NB = tbl_ref.shape[0]
    NN = nmask_ref.shape[1]
    NA = amaskc_ref.shape[1]
    KT = P // _TN
    AT = NA // _TA
    S2 = 1 + KT
    S3 = S2 + 1
    SF = S3 + AT - 1

    @pl.when(i == 0)
    def _gather():
        copies = []
        for b in range(B):
            s = jnp.sum(amask_ref[b, :])
            len_b = jnp.maximum(s, 1) - 1
            copies.append(pltpu.make_async_copy(
                hid_ref.at[b, pl.ds(len_b, 1), :],
                pooled.at[pl.ds(b, 1), :], sem))
        for c in copies:
            c.start()
        for c in copies:
            c.wait()

    @pl.when((i >= 1) & (i < S2))
    def _mlp():
        x1_t = _silu(jnp.dot(pooled[...], w1_ref[...],
                             preferred_element_type=jnp.float32,
                             precision=jax.lax.Precision.DEFAULT))
        part = jnp.dot(x1_t, w2_ref[...], preferred_element_type=jnp.float32,
                       precision=jax.lax.Precision.DEFAULT)

        @pl.when(i == 1)
        def _():
            f_acc[...] = part

        @pl.when(i > 1)
        def _():
            f_acc[...] += part

    @pl.when(i == S2)
    def _name():
        onehot = jnp.where(jax.lax.broadcasted_iota(jnp.int32, (B, NB), 1)
                           == bids_ref[...], 1.0, 0.0).astype(jnp.float32)
        adpt = jnp.dot(onehot, tbl_ref[...],
                       preferred_element_type=jnp.float32)
        A = _silu(f_acc[...]) + scale_ref[0, 0] * adpt
        adapted[...] = A
        nlog[...] = jnp.dot(A, nw_ref[...],
                            preferred_element_type=jnp.float32,
                            precision=jax.lax.Precision.DEFAULT)

    for t in range(AT):
        @pl.when(i == S3 + t)
        def _arg(t=t):
            alog[:, t * _TA:(t + 1) * _TA] = jnp.dot(
                adapted[...], aw_ref[...], preferred_element_type=jnp.float32,
                precision=jax.lax.Precision.DEFAULT)

    @pl.when(i == SF)
    def _finish():
        def head_loss(logits, bias_ref, mask_ref, tgt_ref, ncls):
            logits = logits + bias_ref[...]
            mf = mask_ref[...]
            anyv = jnp.max(mf, axis=1, keepdims=True)
            eff = jnp.maximum(mf, 1.0 - anyv)
            lm = eff * logits + (1.0 - eff) * _LOGIT_FLOOR
            mx = jnp.max(lm, axis=1, keepdims=True)
            lse = jnp.log(jnp.sum(jnp.exp(lm - mx), axis=1)) + mx[:, 0]
            oh = jnp.where(jax.lax.broadcasted_iota(jnp.int32, (B, ncls), 1)
                           == tgt_ref[...], 1.0, 0.0).astype(jnp.float32)
            tgt = jnp.sum(lm * oh, axis=1)
            return jnp.mean(lse - tgt)

        nl = head_loss(nlog[...], nb_ref, nmask_ref, tname_ref, NN)
        al = head_loss(alog[...], ab_ref, amaskc_ref, targ_ref, NA)
        out_ref[0, 0] = nl + al


def kernel(hidden_states, W1, W2, adapter_table, adapter_scale, name_W,
           name_b, arg_W, arg_b, attention_mask, benchmark_ids,
           target_name_ids, target_argument_ids, name_candidate_masks,
           argument_candidate_masks):
    B, S, H = hidden_states.shape
    P = W1.shape[1]
    NB, _ = adapter_table.shape
    NN = name_W.shape[1]
    NA = arg_W.shape[1]
    KT = P // _TN
    AT = NA // _TA
    S2 = 1 + KT
    S3 = S2 + 1
    nsteps = S3 + AT

    out = pl.pallas_call(
        _mega_kernel,
        grid=(nsteps,),
        in_specs=[
            pl.BlockSpec(memory_space=pl.ANY),
            pl.BlockSpec((B, S), lambda i: (0, 0)),
            pl.BlockSpec((H, _TN), lambda i: (0, jnp.clip(i - 1, 0, KT - 1))),
            pl.BlockSpec((_TN, P), lambda i: (jnp.clip(i - 1, 0, KT - 1), 0)),
            pl.BlockSpec((NB, P), lambda i: (0, 0)),
            pl.BlockSpec((1, 1), lambda i: (0, 0)),
            pl.BlockSpec((B, 1), lambda i: (0, 0)),
            pl.BlockSpec((P, NN), lambda i: (0, 0)),
            pl.BlockSpec((1, NN), lambda i: (0, 0)),
            pl.BlockSpec((P, _TA), lambda i: (0, jnp.clip(i - S3, 0, AT - 1))),
            pl.BlockSpec((1, NA), lambda i: (0, 0)),
            pl.BlockSpec((B, NN), lambda i: (0, 0)),
            pl.BlockSpec((B, NA), lambda i: (0, 0)),
            pl.BlockSpec((B, 1), lambda i: (0, 0)),
            pl.BlockSpec((B, 1), lambda i: (0, 0)),
        ],
        out_specs=pl.BlockSpec(memory_space=pltpu.MemorySpace.SMEM),
        out_shape=jax.ShapeDtypeStruct((1, 1), jnp.float32),
        scratch_shapes=[
            pltpu.VMEM((B, H), jnp.float32),
            pltpu.VMEM((B, P), jnp.float32),
            pltpu.VMEM((B, P), jnp.float32),
            pltpu.VMEM((B, NN), jnp.float32),
            pltpu.VMEM((B, NA), jnp.float32),
            pltpu.SemaphoreType.DMA,
        ],
    )(
        hidden_states,
        attention_mask.astype(jnp.int32),
        W1, W2, adapter_table,
        adapter_scale.reshape(1, 1).astype(jnp.float32),
        benchmark_ids.reshape(B, 1).astype(jnp.int32),
        name_W, name_b.reshape(1, NN),
        arg_W, arg_b.reshape(1, NA),
        name_candidate_masks.astype(jnp.float32),
        argument_candidate_masks.astype(jnp.float32),
        target_name_ids.reshape(B, 1).astype(jnp.int32),
        target_argument_ids.reshape(B, 1).astype(jnp.int32),
    )
    return out[0, 0]


# speedup vs baseline: 1.3387x; 1.1210x over previous
import jax
import jax.numpy as jnp
from jax.experimental import pallas as pl
from jax.experimental.pallas import tpu as pltpu

_LOGIT_FLOOR = -1000000000.0
_TK = 256


def _silu(x):
    return x * jax.nn.sigmoid(x)


def _kernel_body(hid_ref, amask_ref, w1_any, w2_any, nw_any, aw_any, tbl_ref,
                 scale_ref, bids_ref, nb_ref, ab_ref, nmask_ref, amaskc_ref,
                 tname_ref, targ_ref, out_ref,
                 w1v, w2v, nwv, awv, pooled,
                 gsem, s1, s2, sn, sa):
    B, S, H = hid_ref.shape
    P = w1_any.shape[1]
    NB = tbl_ref.shape[0]
    NN = nmask_ref.shape[1]
    NA = amaskc_ref.shape[1]
    KT = H // _TK

    gathers = []
    for b in range(B):
        s = jnp.sum(amask_ref[b, :])
        len_b = jnp.maximum(s, 1) - 1
        gathers.append(pltpu.make_async_copy(
            hid_ref.at[b, pl.ds(len_b, 1), :],
            pooled.at[pl.ds(b, 1), :], gsem))
    for c in gathers:
        c.start()

    def slice_copies(src, dst, sems):
        cs = []
        for k in range(KT):
            cs.append(pltpu.make_async_copy(
                src.at[pl.ds(k * _TK, _TK), :],
                dst.at[pl.ds(k * _TK, _TK), :], sems.at[k]))
        return cs

    c1 = slice_copies(w1_any, w1v, s1)
    c2 = slice_copies(w2_any, w2v, s2)
    cn = slice_copies(nw_any, nwv, sn)
    ca = slice_copies(aw_any, awv, sa)
    for k in range(KT):
        c1[k].start()
        c2[k].start()
    for k in range(KT):
        cn[k].start()
        ca[k].start()

    for c in gathers:
        c.wait()
    pooled_v = pooled[...]

    def ksum(acts, copies, wv, ncols):
        acc = jnp.zeros((B, ncols), dtype=jnp.float32)
        for k in range(KT):
            copies[k].wait()
            acc += jnp.dot(acts[:, k * _TK:(k + 1) * _TK],
                           wv[k * _TK:(k + 1) * _TK, :],
                           preferred_element_type=jnp.float32)
        return acc

    x1 = _silu(ksum(pooled_v, c1, w1v, P))
    f = _silu(ksum(x1, c2, w2v, P))

    onehot = jnp.where(jax.lax.broadcasted_iota(jnp.int32, (B, NB), 1)
                       == bids_ref[...], 1.0, 0.0).astype(jnp.float32)
    adpt = jnp.dot(onehot, tbl_ref[...], preferred_element_type=jnp.float32)
    A = f + scale_ref[0, 0] * adpt

    nlog = ksum(A, cn, nwv, NN)
    alog = ksum(A, ca, awv, NA)

    def head_loss(logits, bias_ref, mask_ref, tgt_ref, ncls):
        logits = logits + bias_ref[...]
        mf = mask_ref[...]
        anyv = jnp.max(mf, axis=1, keepdims=True)
        eff = jnp.maximum(mf, 1.0 - anyv)
        lm = eff * logits + (1.0 - eff) * _LOGIT_FLOOR
        mx = jnp.max(lm, axis=1, keepdims=True)
        lse = jnp.log(jnp.sum(jnp.exp(lm - mx), axis=1)) + mx[:, 0]
        oh = jnp.where(jax.lax.broadcasted_iota(jnp.int32, (B, ncls), 1)
                       == tgt_ref[...], 1.0, 0.0).astype(jnp.float32)
        tgt = jnp.sum(lm * oh, axis=1)
        return jnp.mean(lse - tgt)

    nl = head_loss(nlog, nb_ref, nmask_ref, tname_ref, NN)
    al = head_loss(alog, ab_ref, amaskc_ref, targ_ref, NA)
    out_ref[0, 0] = nl + al


def kernel(hidden_states, W1, W2, adapter_table, adapter_scale, name_W,
           name_b, arg_W, arg_b, attention_mask, benchmark_ids,
           target_name_ids, target_argument_ids, name_candidate_masks,
           argument_candidate_masks):
    B, S, H = hidden_states.shape
    P = W1.shape[1]
    NB, _ = adapter_table.shape
    NN = name_W.shape[1]
    NA = arg_W.shape[1]
    KT = H // _TK

    any_spec = pl.BlockSpec(memory_space=pl.ANY)
    out = pl.pallas_call(
        _kernel_body,
        in_specs=[
            any_spec,
            pl.BlockSpec((B, S), lambda: (0, 0)),
            any_spec, any_spec, any_spec, any_spec,
            pl.BlockSpec((NB, P), lambda: (0, 0)),
            pl.BlockSpec((1, 1), lambda: (0, 0)),
            pl.BlockSpec((B, 1), lambda: (0, 0)),
            pl.BlockSpec((1, NN), lambda: (0, 0)),
            pl.BlockSpec((1, NA), lambda: (0, 0)),
            pl.BlockSpec((B, NN), lambda: (0, 0)),
            pl.BlockSpec((B, NA), lambda: (0, 0)),
            pl.BlockSpec((B, 1), lambda: (0, 0)),
            pl.BlockSpec((B, 1), lambda: (0, 0)),
        ],
        out_specs=pl.BlockSpec(memory_space=pltpu.MemorySpace.SMEM),
        out_shape=jax.ShapeDtypeStruct((1, 1), jnp.float32),
        scratch_shapes=[
            pltpu.VMEM((H, P), jnp.float32),
            pltpu.VMEM((P, P), jnp.float32),
            pltpu.VMEM((P, NN), jnp.float32),
            pltpu.VMEM((P, NA), jnp.float32),
            pltpu.VMEM((B, H), jnp.float32),
            pltpu.SemaphoreType.DMA,
            pltpu.SemaphoreType.DMA((KT,)),
            pltpu.SemaphoreType.DMA((KT,)),
            pltpu.SemaphoreType.DMA((KT,)),
            pltpu.SemaphoreType.DMA((KT,)),
        ],
    )(
        hidden_states,
        attention_mask.astype(jnp.int32),
        W1, W2, name_W, arg_W, adapter_table,
        adapter_scale.reshape(1, 1).astype(jnp.float32),
        benchmark_ids.reshape(B, 1).astype(jnp.int32),
        name_b.reshape(1, NN),
        arg_b.reshape(1, NA),
        name_candidate_masks.astype(jnp.float32),
        argument_candidate_masks.astype(jnp.float32),
        target_name_ids.reshape(B, 1).astype(jnp.int32),
        target_argument_ids.reshape(B, 1).astype(jnp.int32),
    )
    return out[0, 0]
